# Initial kernel scaffold; baseline (speedup 1.0000x reference)
import jax
import jax.numpy as jnp
from jax.experimental import pallas as pl


def kernel(i, j, x, W, W_tilde, bias, bias_tilde):
    raise NotImplementedError("write your pallas kernel here")


# speedup vs baseline: 1.0883x; 1.0883x over previous
import functools

import jax
import jax.numpy as jnp
from jax import lax
from jax.experimental import pallas as pl
from jax.experimental.pallas import tpu as pltpu
from jax.experimental.pallas import tpu_sc as plsc

VOCAB = 100000
DIM = 128
BATCH = 16384
X_MAX = 100.0
ALPHA = 0.75

NC = 2
NS = 16
L = 16
NW = NC * NS
BPW = BATCH // NW
CH = 128
NCHUNK = BPW // CH

_mesh = plsc.VectorSubcoreMesh(core_axis_name="c", subcore_axis_name="s")


_GATHER_DNUMS = lax.GatherDimensionNumbers(
    offset_dims=(), collapsed_slice_dims=(0,), start_index_map=(0,))


def _lane_perm(v, perm):
    return lax.gather(v, perm[:, None], _GATHER_DNUMS, (1,),
                      mode=lax.GatherScatterMode.PROMISE_IN_BOUNDS)


def _lane_sum_16(vecs):
    lane = lax.iota(jnp.int32, L)
    s = 1
    while len(vecs) > 1:
        perm = jnp.bitwise_xor(lane, s)
        m = jnp.bitwise_and(lane, s) == 0
        nxt = []
        for k in range(0, len(vecs), 2):
            a, b = vecs[k], vecs[k + 1]
            ar = _lane_perm(a, perm)
            br = _lane_perm(b, perm)
            nxt.append(jnp.where(m, a, br) + jnp.where(m, ar, b))
        vecs = nxt
        s *= 2
    return vecs[0]


@functools.partial(
    pl.kernel,
    out_type=jax.ShapeDtypeStruct((NW, L), jnp.float32),
    mesh=_mesh,
    scratch_types=[
        pltpu.VMEM((CH,), jnp.int32),
        pltpu.VMEM((CH,), jnp.int32),
        pltpu.VMEM((CH, DIM), jnp.float32),
        pltpu.VMEM((CH, DIM), jnp.float32),
        pltpu.VMEM((CH,), jnp.float32),
        pltpu.VMEM((CH,), jnp.float32),
        pltpu.VMEM((CH,), jnp.float32),
        pltpu.VMEM((CH,), jnp.float32),
        pltpu.VMEM((L,), jnp.float32),
        pltpu.SemaphoreType.DMA,
    ],
)
def _glove_sc(i_hbm, j_hbm, a_hbm, wf_hbm, w_hbm, wt_hbm, b_hbm, bt_hbm,
              out_hbm, ii, jj, wi, wj, bi, bj, av, wfv, accbuf, sem):
    wid = lax.axis_index("s") * NC + lax.axis_index("c")
    base = wid * BPW
    acc = jnp.zeros((L,), jnp.float32)
    for ch in range(NCHUNK):
        off = base + ch * CH
        pltpu.sync_copy(i_hbm.at[pl.ds(off, CH)], ii)
        pltpu.sync_copy(j_hbm.at[pl.ds(off, CH)], jj)
        cw = pltpu.async_copy(w_hbm.at[ii], wi, sem)
        cwt = pltpu.async_copy(wt_hbm.at[jj], wj, sem)
        cb = pltpu.async_copy(b_hbm.at[ii], bi, sem)
        cbt = pltpu.async_copy(bt_hbm.at[jj], bj, sem)
        pltpu.sync_copy(a_hbm.at[pl.ds(off, CH)], av)
        pltpu.sync_copy(wf_hbm.at[pl.ds(off, CH)], wfv)
        cw.wait()
        cwt.wait()
        cb.wait()
        cbt.wait()

        def grp_body(g, carry):
            ds = []
            for r16 in range(L):
                r = g * L + r16
                d = wi[r, pl.ds(0, L)] * wj[r, pl.ds(0, L)]
                for k in range(1, DIM // L):
                    d = d + wi[r, pl.ds(k * L, L)] * wj[r, pl.ds(k * L, L)]
                ds.append(d)
            dotv = _lane_sum_16(ds)
            s = pl.ds(g * L, L)
            diff = dotv + bi[s] + bj[s] - av[s]
            return carry + wfv[s] * diff * diff

        acc = lax.fori_loop(0, CH // L, grp_body, acc)
    accbuf[...] = acc
    pltpu.sync_copy(accbuf, out_hbm.at[wid])


def kernel(i, j, x, W, W_tilde, bias, bias_tilde):
    xf = x.astype(jnp.float32)
    a = jnp.log(xf)
    wf = jnp.clip(jnp.power(xf / X_MAX, ALPHA), 0.0, 1.0).astype(jnp.float32)
    parts = _glove_sc(i.astype(jnp.int32), j.astype(jnp.int32), a, wf,
                      W, W_tilde, bias, bias_tilde)
    return jnp.sum(parts) / BATCH


# speedup vs baseline: 1.1498x; 1.0565x over previous
import functools

import jax
import jax.numpy as jnp
from jax import lax
from jax.experimental import pallas as pl
from jax.experimental.pallas import tpu as pltpu
from jax.experimental.pallas import tpu_sc as plsc

VOCAB = 100000
DIM = 128
BATCH = 16384
X_MAX = 100.0
ALPHA = 0.75

NC = 2
NS = 16
L = 16
NW = NC * NS
BPW = BATCH // NW
CH = 128
NCHUNK = BPW // CH

_mesh = plsc.VectorSubcoreMesh(core_axis_name="c", subcore_axis_name="s")


_GATHER_DNUMS = lax.GatherDimensionNumbers(
    offset_dims=(), collapsed_slice_dims=(0,), start_index_map=(0,))


def _lane_perm(v, perm):
    return lax.gather(v, perm[:, None], _GATHER_DNUMS, (1,),
                      mode=lax.GatherScatterMode.PROMISE_IN_BOUNDS)


def _stage_consts():
    lane = lax.iota(jnp.int32, L)
    out = []
    for lvl in range(4):
        s = 1 << lvl
        out.append((jnp.bitwise_xor(lane, s),
                    jnp.bitwise_and(lane, s) == 0))
    return out


def _combine(a, b, perm, m):
    ar = _lane_perm(a, perm)
    br = _lane_perm(b, perm)
    return jnp.where(m, a, br) + jnp.where(m, ar, b)


@functools.partial(
    pl.kernel,
    out_type=jax.ShapeDtypeStruct((NW, L), jnp.float32),
    mesh=_mesh,
    scratch_types=(
        [pltpu.VMEM((CH,), jnp.int32)] * 4
        + [pltpu.VMEM((CH, DIM), jnp.float32)] * 4
        + [pltpu.VMEM((CH,), jnp.float32)] * 8
        + [pltpu.VMEM((L,), jnp.float32)]
        + [pltpu.SemaphoreType.DMA] * 2
    ),
)
def _glove_sc(i_hbm, j_hbm, a_hbm, wf_hbm, w_hbm, wt_hbm, b_hbm, bt_hbm,
              out_hbm, ii0, jj0, ii1, jj1, wi0, wj0, wi1, wj1,
              bi0, bj0, av0, wf0, bi1, bj1, av1, wf1, accbuf, sem0, sem1):
    wid = lax.axis_index("s") * NC + lax.axis_index("c")
    base = wid * BPW
    stages = _stage_consts()
    slots = (
        (ii0, jj0, wi0, wj0, bi0, bj0, av0, wf0, sem0),
        (ii1, jj1, wi1, wj1, bi1, bj1, av1, wf1, sem1),
    )

    def issue(ch):
        ii, jj, wi, wj, bi, bj, av, wfv, sem = slots[ch % 2]
        off = base + ch * CH
        pltpu.sync_copy(i_hbm.at[pl.ds(off, CH)], ii)
        pltpu.sync_copy(j_hbm.at[pl.ds(off, CH)], jj)
        cops = (pltpu.async_copy(w_hbm.at[ii], wi, sem),
                pltpu.async_copy(wt_hbm.at[jj], wj, sem),
                pltpu.async_copy(b_hbm.at[ii], bi, sem),
                pltpu.async_copy(bt_hbm.at[jj], bj, sem))
        pltpu.sync_copy(a_hbm.at[pl.ds(off, CH)], av)
        pltpu.sync_copy(wf_hbm.at[pl.ds(off, CH)], wfv)
        return cops

    acc = jnp.zeros((L,), jnp.float32)
    inflight = issue(0)
    for ch in range(NCHUNK):
        for c in inflight:
            c.wait()
        if ch + 1 < NCHUNK:
            inflight = issue(ch + 1)
        _, _, wi, wj, bi, bj, av, wfv, _ = slots[ch % 2]

        def grp_body(g, carry):
            pending = [None, None, None, None, None]
            for r16 in range(L):
                r = g * L + r16
                d = wi[r, pl.ds(0, L)] * wj[r, pl.ds(0, L)]
                for k in range(1, DIM // L):
                    d = d + wi[r, pl.ds(k * L, L)] * wj[r, pl.ds(k * L, L)]
                lvl = 0
                while pending[lvl] is not None:
                    d = _combine(pending[lvl], d, *stages[lvl])
                    pending[lvl] = None
                    lvl += 1
                pending[lvl] = d
            dotv = pending[4]
            s = pl.ds(g * L, L)
            diff = dotv + bi[s] + bj[s] - av[s]
            return carry + wfv[s] * diff * diff

        acc = lax.fori_loop(0, CH // L, grp_body, acc)
    accbuf[...] = acc
    pltpu.sync_copy(accbuf, out_hbm.at[wid])


def kernel(i, j, x, W, W_tilde, bias, bias_tilde):
    xf = x.astype(jnp.float32)
    a = jnp.log(xf)
    wf = jnp.clip(jnp.power(xf / X_MAX, ALPHA), 0.0, 1.0).astype(jnp.float32)
    parts = _glove_sc(i.astype(jnp.int32), j.astype(jnp.int32), a, wf,
                      W, W_tilde, bias, bias_tilde)
    return jnp.sum(parts) / BATCH


# speedup vs baseline: 1.4126x; 1.2285x over previous
import functools

import jax
import jax.numpy as jnp
from jax import lax
from jax.experimental import pallas as pl
from jax.experimental.pallas import tpu as pltpu
from jax.experimental.pallas import tpu_sc as plsc

VOCAB = 100000
DIM = 128
BATCH = 16384
X_MAX = 100.0
ALPHA = 0.75

NC = 2
NS = 16
L = 16
NW = NC * NS
BPW = BATCH // NW
CH = 128
NPAIR = BPW // (2 * CH)

_mesh = plsc.VectorSubcoreMesh(core_axis_name="c", subcore_axis_name="s")

_GATHER_DNUMS = lax.GatherDimensionNumbers(
    offset_dims=(), collapsed_slice_dims=(0,), start_index_map=(0,))


def _lane_perm(v, perm):
    return lax.gather(v, perm[:, None], _GATHER_DNUMS, (1,),
                      mode=lax.GatherScatterMode.PROMISE_IN_BOUNDS)


def _stage_consts():
    lane = lax.iota(jnp.int32, L)
    out = []
    for lvl in range(4):
        s = 1 << lvl
        out.append((jnp.bitwise_xor(lane, s),
                    jnp.bitwise_and(lane, s) == 0))
    return out


def _combine(a, b, perm, m):
    ar = _lane_perm(a, perm)
    br = _lane_perm(b, perm)
    return jnp.where(m, a, br) + jnp.where(m, ar, b)


@functools.partial(
    pl.kernel,
    out_type=jax.ShapeDtypeStruct((NW, L), jnp.float32),
    mesh=_mesh,
    scratch_types=(
        [pltpu.VMEM((BPW,), jnp.int32)] * 2
        + [pltpu.VMEM((BPW,), jnp.float32)] * 2
        + [pltpu.VMEM((CH, DIM), jnp.float32)] * 4
        + [pltpu.VMEM((CH,), jnp.float32)] * 4
        + [pltpu.VMEM((L,), jnp.float32)]
        + [pltpu.SemaphoreType.DMA] * 2
    ),
)
def _glove_sc(i_hbm, j_hbm, a_hbm, wf_hbm, w_hbm, wt_hbm, b_hbm, bt_hbm,
              out_hbm, ii_all, jj_all, av_all, wf_all, wi0, wi1, wj0, wj1,
              bi0, bj0, bi1, bj1, accbuf, sem0, sem1):
    wid = lax.axis_index("s") * NC + lax.axis_index("c")
    base = wid * BPW
    stages = _stage_consts()
    pltpu.sync_copy(i_hbm.at[pl.ds(base, BPW)], ii_all)
    pltpu.sync_copy(j_hbm.at[pl.ds(base, BPW)], jj_all)
    pltpu.sync_copy(a_hbm.at[pl.ds(base, BPW)], av_all)
    pltpu.sync_copy(wf_hbm.at[pl.ds(base, BPW)], wf_all)
    slots = ((wi0, wj0, bi0, bj0, sem0), (wi1, wj1, bi1, bj1, sem1))

    def issue(loc, slot):
        wi, wj, bi, bj, sem = slots[slot]
        iref = ii_all.at[pl.ds(loc, CH)]
        jref = jj_all.at[pl.ds(loc, CH)]
        pltpu.async_copy(w_hbm.at[iref], wi, sem)
        pltpu.async_copy(wt_hbm.at[jref], wj, sem)
        pltpu.async_copy(b_hbm.at[iref], bi, sem)
        pltpu.async_copy(bt_hbm.at[jref], bj, sem)

    def drain(slot):
        wi, wj, bi, bj, sem = slots[slot]
        iref = ii_all.at[pl.ds(0, CH)]
        jref = jj_all.at[pl.ds(0, CH)]
        pltpu.make_async_copy(w_hbm.at[iref], wi, sem).wait()
        pltpu.make_async_copy(wt_hbm.at[jref], wj, sem).wait()
        pltpu.make_async_copy(b_hbm.at[iref], bi, sem).wait()
        pltpu.make_async_copy(bt_hbm.at[jref], bj, sem).wait()

    def compute(loc, slot, acc):
        wi, wj, bi, bj, _ = slots[slot]

        def grp_body(g, carry):
            pending = [None, None, None, None, None]
            for r16 in range(L):
                r = g * L + r16
                d = wi[r, pl.ds(0, L)] * wj[r, pl.ds(0, L)]
                for k in range(1, DIM // L):
                    d = d + wi[r, pl.ds(k * L, L)] * wj[r, pl.ds(k * L, L)]
                lvl = 0
                while pending[lvl] is not None:
                    d = _combine(pending[lvl], d, *stages[lvl])
                    pending[lvl] = None
                    lvl += 1
                pending[lvl] = d
            dotv = pending[4]
            sl = pl.ds(g * L, L)
            sg = pl.ds(loc + g * L, L)
            diff = dotv + bi[sl] + bj[sl] - av_all[sg]
            return carry + wf_all[sg] * diff * diff

        return lax.fori_loop(0, CH // L, grp_body, acc, unroll=1)

    issue(0, 0)

    def pair_body(k, acc):
        loc0 = 2 * k * CH
        drain(0)
        issue(loc0 + CH, 1)
        acc = compute(loc0, 0, acc)
        drain(1)

        @pl.when(k + 1 < NPAIR)
        def _():
            issue(loc0 + 2 * CH, 0)

        return compute(loc0 + CH, 1, acc)

    acc = lax.fori_loop(0, NPAIR, pair_body, jnp.zeros((L,), jnp.float32),
                        unroll=1)
    accbuf[...] = acc
    pltpu.sync_copy(accbuf, out_hbm.at[wid])


def kernel(i, j, x, W, W_tilde, bias, bias_tilde):
    xf = x.astype(jnp.float32)
    a = jnp.log(xf)
    wf = jnp.clip(jnp.power(xf / X_MAX, ALPHA), 0.0, 1.0).astype(jnp.float32)
    parts = _glove_sc(i.astype(jnp.int32), j.astype(jnp.int32), a, wf,
                      W, W_tilde, bias, bias_tilde)
    return jnp.sum(parts) / BATCH


# speedup vs baseline: 1.4679x; 1.0392x over previous
import functools

import jax
import jax.numpy as jnp
from jax import lax
from jax.experimental import pallas as pl
from jax.experimental.pallas import tpu as pltpu
from jax.experimental.pallas import tpu_sc as plsc

VOCAB = 100000
DIM = 128
BATCH = 16384
X_MAX = 100.0
ALPHA = 0.75

NC = 2
NS = 16
L = 16
NW = NC * NS
BPW = BATCH // NW
CH = 128
NCHUNK = BPW // CH

_mesh = plsc.VectorSubcoreMesh(core_axis_name="c", subcore_axis_name="s")

_GATHER_DNUMS = lax.GatherDimensionNumbers(
    offset_dims=(), collapsed_slice_dims=(0,), start_index_map=(0,))


def _lane_perm(v, perm):
    return lax.gather(v, perm[:, None], _GATHER_DNUMS, (1,),
                      mode=lax.GatherScatterMode.PROMISE_IN_BOUNDS)


def _stage_consts():
    lane = lax.iota(jnp.int32, L)
    out = []
    for lvl in range(4):
        s = 1 << lvl
        out.append((jnp.bitwise_xor(lane, s),
                    jnp.bitwise_and(lane, s) == 0))
    return out


def _combine(a, b, perm, m):
    ar = _lane_perm(a, perm)
    br = _lane_perm(b, perm)
    return jnp.where(m, a, br) + jnp.where(m, ar, b)


@functools.partial(
    pl.kernel,
    out_type=jax.ShapeDtypeStruct((NW, L), jnp.float32),
    mesh=_mesh,
    scratch_types=(
        [pltpu.VMEM((BPW,), jnp.int32)] * 2
        + [pltpu.VMEM((BPW,), jnp.float32)] * 2
        + [pltpu.VMEM((2 * CH, DIM), jnp.float32)] * 2
        + [pltpu.VMEM((2 * CH,), jnp.float32)] * 2
        + [pltpu.VMEM((L,), jnp.float32)]
        + [pltpu.SemaphoreType.DMA] * 2
    ),
)
def _glove_sc(i_hbm, j_hbm, a_hbm, wf_hbm, w_hbm, wt_hbm, b_hbm, bt_hbm,
              out_hbm, ii_all, jj_all, av_all, wf_all, wi2, wj2,
              bi2, bj2, accbuf, sem0, sem1):
    wid = lax.axis_index("s") * NC + lax.axis_index("c")
    base = wid * BPW
    stages = _stage_consts()
    pltpu.sync_copy(i_hbm.at[pl.ds(base, BPW)], ii_all)
    pltpu.sync_copy(j_hbm.at[pl.ds(base, BPW)], jj_all)
    pltpu.sync_copy(a_hbm.at[pl.ds(base, BPW)], av_all)
    pltpu.sync_copy(wf_hbm.at[pl.ds(base, BPW)], wf_all)
    sems = (sem0, sem1)

    def issue(loc, slot):
        soff = slot * CH
        iref = ii_all.at[pl.ds(loc, CH)]
        jref = jj_all.at[pl.ds(loc, CH)]
        sem = sems[slot]
        pltpu.async_copy(w_hbm.at[iref], wi2.at[pl.ds(soff, CH), :], sem)
        pltpu.async_copy(wt_hbm.at[jref], wj2.at[pl.ds(soff, CH), :], sem)
        pltpu.async_copy(b_hbm.at[iref], bi2.at[pl.ds(soff, CH)], sem)
        pltpu.async_copy(bt_hbm.at[jref], bj2.at[pl.ds(soff, CH)], sem)

    def drain(slot):
        soff = slot * CH
        iref = ii_all.at[pl.ds(0, CH)]
        jref = jj_all.at[pl.ds(0, CH)]
        sem = sems[slot]
        pltpu.make_async_copy(w_hbm.at[iref], wi2.at[pl.ds(soff, CH), :],
                              sem).wait()
        pltpu.make_async_copy(wt_hbm.at[jref], wj2.at[pl.ds(soff, CH), :],
                              sem).wait()
        pltpu.make_async_copy(b_hbm.at[iref], bi2.at[pl.ds(soff, CH)],
                              sem).wait()
        pltpu.make_async_copy(bt_hbm.at[jref], bj2.at[pl.ds(soff, CH)],
                              sem).wait()

    issue(0, 0)

    def chunk_body(ch, acc):
        par = jnp.bitwise_and(ch, 1)
        loc = ch * CH
        soff = par * CH

        @pl.when(par == 0)
        def _():
            drain(0)

        @pl.when(par == 1)
        def _():
            drain(1)

        @pl.when(ch + 1 < NCHUNK)
        def _():
            @pl.when(par == 0)
            def _():
                issue(loc + CH, 1)

            @pl.when(par == 1)
            def _():
                issue(loc + CH, 0)

        def grp_body(g, carry):
            pending = [None, None, None, None, None]
            for r16 in range(L):
                r = soff + g * L + r16
                d = wi2[r, pl.ds(0, L)] * wj2[r, pl.ds(0, L)]
                for k in range(1, DIM // L):
                    d = d + wi2[r, pl.ds(k * L, L)] * wj2[r, pl.ds(k * L, L)]
                lvl = 0
                while pending[lvl] is not None:
                    d = _combine(pending[lvl], d, *stages[lvl])
                    pending[lvl] = None
                    lvl += 1
                pending[lvl] = d
            dotv = pending[4]
            sl = pl.ds(soff + g * L, L)
            sg = pl.ds(loc + g * L, L)
            diff = dotv + bi2[sl] + bj2[sl] - av_all[sg]
            return carry + wf_all[sg] * diff * diff

        return lax.fori_loop(0, CH // L, grp_body, acc, unroll=1)

    acc = lax.fori_loop(0, NCHUNK, chunk_body, jnp.zeros((L,), jnp.float32),
                        unroll=1)
    accbuf[...] = acc
    pltpu.sync_copy(accbuf, out_hbm.at[wid])


def kernel(i, j, x, W, W_tilde, bias, bias_tilde):
    xf = x.astype(jnp.float32)
    a = jnp.log(xf)
    wf = jnp.clip(jnp.power(xf / X_MAX, ALPHA), 0.0, 1.0).astype(jnp.float32)
    parts = _glove_sc(i.astype(jnp.int32), j.astype(jnp.int32), a, wf,
                      W, W_tilde, bias, bias_tilde)
    return jnp.sum(parts) / BATCH


# speedup vs baseline: 1.6574x; 1.1290x over previous
import functools

import jax
import jax.numpy as jnp
from jax import lax
from jax.experimental import pallas as pl
from jax.experimental.pallas import tpu as pltpu
from jax.experimental.pallas import tpu_sc as plsc

VOCAB = 100000
DIM = 128
BATCH = 16384
X_MAX = 100.0
ALPHA = 0.75

NC = 2
NS = 16
L = 16
NW = NC * NS
BPW = BATCH // NW
CH = 128
NCHUNK = BPW // CH

_mesh = plsc.VectorSubcoreMesh(core_axis_name="c", subcore_axis_name="s")

_GATHER_DNUMS = lax.GatherDimensionNumbers(
    offset_dims=(), collapsed_slice_dims=(0,), start_index_map=(0,))


def _lane_perm(v, perm):
    return lax.gather(v, perm[:, None], _GATHER_DNUMS, (1,),
                      mode=lax.GatherScatterMode.PROMISE_IN_BOUNDS)


def _stage_consts():
    lane = lax.iota(jnp.int32, L)
    out = []
    for lvl in range(4):
        s = 1 << lvl
        out.append((jnp.bitwise_xor(lane, s),
                    jnp.bitwise_and(lane, s) == 0))
    return out


def _combine(a, b, perm, m):
    ar = _lane_perm(a, perm)
    br = _lane_perm(b, perm)
    return jnp.where(m, a, br) + jnp.where(m, ar, b)


def _lane_sum_16(vecs, stages):
    lvl = 0
    while len(vecs) > 1:
        perm, m = stages[lvl]
        vecs = [_combine(vecs[k], vecs[k + 1], perm, m)
                for k in range(0, len(vecs), 2)]
        lvl += 1
    return vecs[0]


@functools.partial(
    pl.kernel,
    out_type=jax.ShapeDtypeStruct((NW, L), jnp.float32),
    mesh=_mesh,
    scratch_types=(
        [pltpu.VMEM((BPW,), jnp.int32)] * 2
        + [pltpu.VMEM((BPW,), jnp.float32)] * 2
        + [pltpu.VMEM((2 * CH, DIM), jnp.float32)] * 2
        + [pltpu.VMEM((2 * CH,), jnp.float32)] * 2
        + [pltpu.VMEM((L * L,), jnp.float32)]
        + [pltpu.VMEM((L,), jnp.float32)]
        + [pltpu.SemaphoreType.DMA] * 2
    ),
)
def _glove_sc(i_hbm, j_hbm, a_hbm, wf_hbm, w_hbm, wt_hbm, b_hbm, bt_hbm,
              out_hbm, ii_all, jj_all, av_all, wf_all, wi2, wj2,
              bi2, bj2, dmat, accbuf, sem0, sem1):
    wid = lax.axis_index("s") * NC + lax.axis_index("c")
    base = wid * BPW
    stages = _stage_consts()
    pltpu.sync_copy(i_hbm.at[pl.ds(base, BPW)], ii_all)
    pltpu.sync_copy(j_hbm.at[pl.ds(base, BPW)], jj_all)
    pltpu.sync_copy(a_hbm.at[pl.ds(base, BPW)], av_all)
    pltpu.sync_copy(wf_hbm.at[pl.ds(base, BPW)], wf_all)
    sems = (sem0, sem1)

    def issue(loc, slot):
        soff = slot * CH
        iref = ii_all.at[pl.ds(loc, CH)]
        jref = jj_all.at[pl.ds(loc, CH)]
        sem = sems[slot]
        pltpu.async_copy(w_hbm.at[iref], wi2.at[pl.ds(soff, CH), :], sem)
        pltpu.async_copy(wt_hbm.at[jref], wj2.at[pl.ds(soff, CH), :], sem)
        pltpu.async_copy(b_hbm.at[iref], bi2.at[pl.ds(soff, CH)], sem)
        pltpu.async_copy(bt_hbm.at[jref], bj2.at[pl.ds(soff, CH)], sem)

    def drain(slot):
        soff = slot * CH
        iref = ii_all.at[pl.ds(0, CH)]
        jref = jj_all.at[pl.ds(0, CH)]
        sem = sems[slot]
        pltpu.make_async_copy(w_hbm.at[iref], wi2.at[pl.ds(soff, CH), :],
                              sem).wait()
        pltpu.make_async_copy(wt_hbm.at[jref], wj2.at[pl.ds(soff, CH), :],
                              sem).wait()
        pltpu.make_async_copy(b_hbm.at[iref], bi2.at[pl.ds(soff, CH)],
                              sem).wait()
        pltpu.make_async_copy(bt_hbm.at[jref], bj2.at[pl.ds(soff, CH)],
                              sem).wait()

    issue(0, 0)

    def chunk_body(ch, acc):
        par = jnp.bitwise_and(ch, 1)
        loc = ch * CH
        soff = par * CH

        @pl.when(par == 0)
        def _():
            drain(0)

        @pl.when(par == 1)
        def _():
            drain(1)

        @pl.when(ch + 1 < NCHUNK)
        def _():
            @pl.when(par == 0)
            def _():
                issue(loc + CH, 1)

            @pl.when(par == 1)
            def _():
                issue(loc + CH, 0)

        def grp_body(g, carry):
            def row_body(r16, t):
                r = soff + g * L + r16
                d0 = wi2[r, pl.ds(0, L)] * wj2[r, pl.ds(0, L)]
                d1 = wi2[r, pl.ds(L, L)] * wj2[r, pl.ds(L, L)]
                d2 = wi2[r, pl.ds(2 * L, L)] * wj2[r, pl.ds(2 * L, L)]
                d3 = wi2[r, pl.ds(3 * L, L)] * wj2[r, pl.ds(3 * L, L)]
                for k in range(4, DIM // L):
                    q = k % 4
                    p = wi2[r, pl.ds(k * L, L)] * wj2[r, pl.ds(k * L, L)]
                    if q == 0:
                        d0 = d0 + p
                    elif q == 1:
                        d1 = d1 + p
                    elif q == 2:
                        d2 = d2 + p
                    else:
                        d3 = d3 + p
                dmat[pl.ds(r16 * L, L)] = (d0 + d1) + (d2 + d3)
                return t

            lax.fori_loop(0, L, row_body, 0, unroll=1)
            dotv = _lane_sum_16([dmat[pl.ds(r * L, L)] for r in range(L)],
                                stages)
            sl = pl.ds(soff + g * L, L)
            sg = pl.ds(loc + g * L, L)
            diff = dotv + bi2[sl] + bj2[sl] - av_all[sg]
            return carry + wf_all[sg] * diff * diff

        return lax.fori_loop(0, CH // L, grp_body, acc, unroll=1)

    acc = lax.fori_loop(0, NCHUNK, chunk_body, jnp.zeros((L,), jnp.float32),
                        unroll=1)
    accbuf[...] = acc
    pltpu.sync_copy(accbuf, out_hbm.at[wid])


def kernel(i, j, x, W, W_tilde, bias, bias_tilde):
    xf = x.astype(jnp.float32)
    a = jnp.log(xf)
    wf = jnp.clip(jnp.power(xf / X_MAX, ALPHA), 0.0, 1.0).astype(jnp.float32)
    parts = _glove_sc(i.astype(jnp.int32), j.astype(jnp.int32), a, wf,
                      W, W_tilde, bias, bias_tilde)
    return jnp.sum(parts) / BATCH
